# Initial kernel scaffold; baseline (speedup 1.0000x reference)
import jax
import jax.numpy as jnp
from jax.experimental import pallas as pl


def kernel(x, edge_index, edge_attr, batch, params):
    raise NotImplementedError("write your pallas kernel here")


# speedup vs baseline: 3.2748x; 3.2748x over previous
import functools

import jax
import jax.numpy as jnp
from jax import lax
from jax.experimental import pallas as pl
from jax.experimental.pallas import tpu as pltpu
import jax.experimental.pallas.tpu_sc as plsc

_N = 10000
_E = 160000
_HID = 128
_G = 64

_NW = 32
_CH = 79
_C = 128
_EP = _CH * _C
_E2P = _NW * _EP
_R = 10240
_STRIPE = _R // 16

_INTERPRET = False


def _full(shape):
    nd = len(shape)
    return pl.BlockSpec(shape, lambda i, _nd=nd: (0,) * _nd)


def _ln(t, g, b):
    m = jnp.mean(t, axis=-1, keepdims=True)
    v = jnp.mean((t - m) ** 2, axis=-1, keepdims=True)
    return (t - m) * lax.rsqrt(v + 1e-5) * g + b


def _edge_mlp_body(ea, W1, b1, g1, be1, W2, b2, out):
    t = jnp.dot(ea[...], W1[...], preferred_element_type=jnp.float32) + b1[...]
    t = jnp.maximum(t, 0.0)
    t = _ln(t, g1[...], be1[...])
    out[...] = jnp.dot(t, W2[...], preferred_element_type=jnp.float32) + b2[...]


def _edge_mlp(ea2, W1p, b1, g1, be1, W2, b2):
    blk = 2048
    grid = _E2P // blk
    return pl.pallas_call(
        _edge_mlp_body,
        grid=(grid,),
        in_specs=[
            pl.BlockSpec((blk, 32), lambda i: (i, 0)),
            _full((32, _HID)), _full((1, _HID)), _full((1, _HID)),
            _full((1, _HID)), _full((_HID, _HID)), _full((1, _HID)),
        ],
        out_specs=pl.BlockSpec((blk, _HID), lambda i: (i, 0)),
        out_shape=jax.ShapeDtypeStruct((_E2P, _HID), jnp.float32),
        interpret=_INTERPRET,
    )(ea2, W1p, b1, g1, be1, W2, b2)


def _node_in_body(x, W, b, out):
    out[...] = jnp.dot(x[...], W[...], preferred_element_type=jnp.float32) + b[...]


def _node_in(x, W, b):
    blk = 2000
    return pl.pallas_call(
        _node_in_body,
        grid=(_N // blk,),
        in_specs=[
            pl.BlockSpec((blk, _HID), lambda i: (i, 0)),
            _full((_HID, _HID)), _full((1, _HID)),
        ],
        out_specs=pl.BlockSpec((blk, _HID), lambda i: (i, 0)),
        out_shape=jax.ShapeDtypeStruct((_N, _HID), jnp.float32),
        interpret=_INTERPRET,
    )(x, W, b)


def _node_update_body(h, a, eps, W1, b1, g1, be1, W2, b2, ng, nb, out):
    hh = h[...]
    agg = a[0] + a[1]
    z = (1.0 + eps[0]) * hh + agg
    t = jnp.dot(z, W1[...], preferred_element_type=jnp.float32) + b1[...]
    t = jnp.maximum(t, 0.0)
    t = _ln(t, g1[...], be1[...])
    z = jnp.dot(t, W2[...], preferred_element_type=jnp.float32) + b2[...]
    z = jnp.maximum(z, 0.0)
    z = _ln(z, ng[...], nb[...])
    out[...] = z + hh


def _node_update(h, aggr, eps, W1, b1, g1, be1, W2, b2, ng, nb):
    blk = 2000
    return pl.pallas_call(
        _node_update_body,
        grid=(_N // blk,),
        in_specs=[
            pl.BlockSpec((blk, _HID), lambda i: (i, 0)),
            pl.BlockSpec((2, blk, _HID), lambda i: (0, i, 0)),
            pl.BlockSpec(memory_space=pltpu.SMEM),
            _full((_HID, _HID)), _full((1, _HID)), _full((1, _HID)),
            _full((1, _HID)), _full((_HID, _HID)), _full((1, _HID)),
            _full((1, _HID)), _full((1, _HID)),
        ],
        out_specs=pl.BlockSpec((blk, _HID), lambda i: (i, 0)),
        out_shape=jax.ShapeDtypeStruct((_N, _HID), jnp.float32),
        interpret=_INTERPRET,
    )(h, aggr, eps, W1, b1, g1, be1, W2, b2, ng, nb)


def _pool_body(h, bc, br, gW1, gb1, gg1, gbe1, gW2, gb2,
               hW1, hb1, hg1, hbe1, hW2, hb2, out):
    hh = h[...]
    t = jnp.dot(hh, gW1[...], preferred_element_type=jnp.float32) + gb1[...]
    t = jnp.maximum(t, 0.0)
    t = _ln(t, gg1[...], gbe1[...])
    gate = jnp.dot(t, gW2[...], preferred_element_type=jnp.float32) + gb2[...]

    oh = lax.broadcasted_iota(jnp.int32, (_N, _G), 1) == bc[...]
    ohf = oh.astype(jnp.float32)
    neg = jnp.float32(-jnp.inf)
    gmax = jnp.max(jnp.where(oh, gate, neg), axis=0, keepdims=True)
    gmax = jnp.where(jnp.isfinite(gmax), gmax, 0.0)
    gpn = jnp.sum(ohf * gmax, axis=1, keepdims=True)
    ex = jnp.exp(gate - gpn)
    den = jnp.sum(ohf * ex, axis=0, keepdims=True)
    dpn = jnp.sum(ohf * den, axis=1, keepdims=True)
    attn = ex / (dpn + 1e-16)
    ah = attn * hh

    oht = (lax.broadcasted_iota(jnp.int32, (_G, _N), 0) == br[...]).astype(jnp.float32)
    g = jnp.dot(oht, ah, preferred_element_type=jnp.float32)

    t2 = jnp.dot(g, hW1[...], preferred_element_type=jnp.float32) + hb1[...]
    t2 = jnp.maximum(t2, 0.0)
    t2 = _ln(t2, hg1[...], hbe1[...])
    out[...] = jnp.dot(t2, hW2[...], preferred_element_type=jnp.float32) + hb2[...]


def _pool(h, bc, br, gW1, gb1, gg1, gbe1, gW2, gb2, hW1, hb1, hg1, hbe1, hW2, hb2):
    return pl.pallas_call(
        _pool_body,
        out_shape=jax.ShapeDtypeStruct((_G, 1), jnp.float32),
        interpret=_INTERPRET,
    )(h, bc, br, gW1, gb1, gg1, gbe1, gW2, gb2, hW1, hb1, hg1, hbe1, hW2, hb2)


def _sc_aggregate(src3, dst3, e, h):
    mesh = plsc.VectorSubcoreMesh(core_axis_name="c", subcore_axis_name="s")

    @functools.partial(
        pl.kernel,
        out_type=jax.ShapeDtypeStruct((2, _R, _HID), jnp.float32),
        mesh=mesh,
        scratch_types=[
            pltpu.VMEM((1, _C), jnp.int32),
            pltpu.VMEM((1, _C), jnp.int32),
            pltpu.VMEM((_C, _HID), jnp.float32),
            pltpu.VMEM((_C, _HID), jnp.float32),
            pltpu.VMEM_SHARED((_R, _HID), jnp.float32),
            pltpu.SemaphoreType.DMA,
        ],
    )
    def k(src_hbm, dst_hbm, e_hbm, h_hbm, out_hbm, srcb, dstb, ebuf, hbuf,
          aggr_sh, sem):
        c = lax.axis_index("c")
        s = lax.axis_index("s")
        wid = s * 2 + c

        zero = jnp.zeros((16,), jnp.float32)

        def zrow(r, carry):
            for kk in range(8):
                ebuf[r, pl.ds(kk * 16, 16)] = zero
            return carry

        lax.fori_loop(0, _C, zrow, 0)
        for b in range(_STRIPE // _C):
            pltpu.sync_copy(ebuf, aggr_sh.at[pl.ds(s * _STRIPE + b * _C, _C)])
        plsc.subcore_barrier()

        def chunk(j, carry):
            base = wid * _EP + j * _C
            pltpu.sync_copy(src_hbm.at[wid, pl.ds(j, 1)], srcb)
            pltpu.sync_copy(dst_hbm.at[wid, pl.ds(j, 1)], dstb)
            pltpu.sync_copy(e_hbm.at[pl.ds(base, _C)], ebuf)
            pltpu.async_copy(h_hbm.at[srcb.at[0]], hbuf, sem).wait()

            def row(r, rc):
                for kk in range(8):
                    sl = pl.ds(kk * 16, 16)
                    hbuf[r, sl] = jnp.maximum(ebuf[r, sl] + hbuf[r, sl], 0.0)
                return rc

            lax.fori_loop(0, _C, row, 0)
            pltpu.sync_copy(hbuf, aggr_sh.at[dstb.at[0]], add=True)
            return carry

        lax.fori_loop(0, _CH, chunk, 0)
        plsc.subcore_barrier()
        pltpu.sync_copy(aggr_sh.at[pl.ds(s * _STRIPE, _STRIPE)],
                        out_hbm.at[c, pl.ds(s * _STRIPE, _STRIPE)])

    return k(src3, dst3, e, h)


def kernel(x, edge_index, edge_attr, batch, params):
    p = params
    f32 = jnp.float32

    dcol = jnp.ones((_E, 1), f32)
    ea2 = jnp.concatenate([
        jnp.concatenate([edge_attr, dcol], axis=1),
        jnp.concatenate([edge_attr, -dcol], axis=1),
    ], axis=0)
    ea2 = jnp.pad(ea2, ((0, _E2P - 2 * _E), (0, 32 - 17)))
    W1p = jnp.pad(p['edge_W1'], ((0, 32 - 17), (0, 0)))

    pad = _E2P - 2 * _E
    src_pad = (jnp.arange(pad, dtype=jnp.int32) * 37) % _N
    dst_pad = _N + (jnp.arange(pad, dtype=jnp.int32) % (_R - _N))
    src = jnp.concatenate([edge_index[0], edge_index[1], src_pad])
    dst = jnp.concatenate([edge_index[1], edge_index[0], dst_pad])
    src3 = src.reshape(_NW, _CH, _C)
    dst3 = dst.reshape(_NW, _CH, _C)

    def row(v):
        return v.reshape(1, -1)

    e = _edge_mlp(ea2, W1p, row(p['edge_b1']), row(p['edge_g1']),
                  row(p['edge_be1']), p['edge_W2'], row(p['edge_b2']))

    h = _node_in(x, p['nin_W'], row(p['nin_b']))

    for l in range(3):
        aggr = _sc_aggregate(src3, dst3, e, h)
        h = _node_update(
            h, aggr, p['conv_eps'][l].reshape(1),
            p['conv_W1'][l], row(p['conv_b1'][l]), row(p['conv_g1'][l]),
            row(p['conv_be1'][l]), p['conv_W2'][l], row(p['conv_b2'][l]),
            row(p['norm_g'][l]), row(p['norm_be'][l]))

    logit = _pool(
        h, batch.reshape(_N, 1), batch.reshape(1, _N),
        p['gate_W1'], row(p['gate_b1']), row(p['gate_g1']), row(p['gate_be1']),
        p['gate_W2'], row(p['gate_b2']),
        p['head_W1'], row(p['head_b1']), row(p['head_g1']), row(p['head_be1']),
        p['head_W2'], row(p['head_b2']))
    return logit.reshape(_G)


# speedup vs baseline: 5.0485x; 1.5416x over previous
import functools

import jax
import jax.numpy as jnp
from jax import lax
from jax.experimental import pallas as pl
from jax.experimental.pallas import tpu as pltpu
import jax.experimental.pallas.tpu_sc as plsc

_N = 10000
_E = 160000
_HID = 128
_G = 64

_NW = 32
_CH = 160
_C = 64
_BI = 16
_EP = _CH * _C
_E2P = _NW * _EP
_R = 10240
_STRIPE = _R // 16

def _full(shape):
    nd = len(shape)
    return pl.BlockSpec(shape, lambda i, _nd=nd: (0,) * _nd)


def _pack_bf16(t):
    bits = lax.bitcast_convert_type(t, jnp.int32)
    r = (bits + 0x7FFF + ((bits >> 16) & 1)) >> 16
    return (r[:, :_HID // 2] & 0xFFFF) | (r[:, _HID // 2:] << 16)


def _ln(t, g, b):
    m = jnp.mean(t, axis=-1, keepdims=True)
    v = jnp.mean((t - m) ** 2, axis=-1, keepdims=True)
    return (t - m) * lax.rsqrt(v + 1e-5) * g + b


def _edge_mlp_body(ea, W1, b1, g1, be1, W2, b2, out):
    t = jnp.dot(ea[...], W1[...], preferred_element_type=jnp.float32) + b1[...]
    t = jnp.maximum(t, 0.0)
    t = _ln(t, g1[...], be1[...])
    t = jnp.dot(t, W2[...], preferred_element_type=jnp.float32) + b2[...]
    out[...] = _pack_bf16(t)


def _edge_mlp(ea2, W1p, b1, g1, be1, W2, b2):
    blk = 2048
    grid = _E2P // blk
    return pl.pallas_call(
        _edge_mlp_body,
        grid=(grid,),
        in_specs=[
            pl.BlockSpec((blk, 32), lambda i: (i, 0)),
            _full((32, _HID)), _full((1, _HID)), _full((1, _HID)),
            _full((1, _HID)), _full((_HID, _HID)), _full((1, _HID)),
        ],
        out_specs=pl.BlockSpec((blk, _HID // 2), lambda i: (i, 0)),
        out_shape=jax.ShapeDtypeStruct((_E2P, _HID // 2), jnp.int32),
    )(ea2, W1p, b1, g1, be1, W2, b2)


def _node_in_body(x, W, b, out):
    out[...] = jnp.dot(x[...], W[...], preferred_element_type=jnp.float32) + b[...]


def _node_in(x, W, b):
    blk = 2000
    return pl.pallas_call(
        _node_in_body,
        grid=(_N // blk,),
        in_specs=[
            pl.BlockSpec((blk, _HID), lambda i: (i, 0)),
            _full((_HID, _HID)), _full((1, _HID)),
        ],
        out_specs=pl.BlockSpec((blk, _HID), lambda i: (i, 0)),
        out_shape=jax.ShapeDtypeStruct((_N, _HID), jnp.float32),
    )(x, W, b)


def _node_update_body(h, a, eps, W1, b1, g1, be1, W2, b2, ng, nb, out):
    hh = h[...]
    agg = a[0] + a[1]
    z = (1.0 + eps[0]) * hh + agg
    t = jnp.dot(z, W1[...], preferred_element_type=jnp.float32) + b1[...]
    t = jnp.maximum(t, 0.0)
    t = _ln(t, g1[...], be1[...])
    z = jnp.dot(t, W2[...], preferred_element_type=jnp.float32) + b2[...]
    z = jnp.maximum(z, 0.0)
    z = _ln(z, ng[...], nb[...])
    out[...] = z + hh


def _node_update(h, aggr, eps, W1, b1, g1, be1, W2, b2, ng, nb):
    blk = 2000
    return pl.pallas_call(
        _node_update_body,
        grid=(_N // blk,),
        in_specs=[
            pl.BlockSpec((blk, _HID), lambda i: (i, 0)),
            pl.BlockSpec((2, blk, _HID), lambda i: (0, i, 0)),
            pl.BlockSpec(memory_space=pltpu.SMEM),
            _full((_HID, _HID)), _full((1, _HID)), _full((1, _HID)),
            _full((1, _HID)), _full((_HID, _HID)), _full((1, _HID)),
            _full((1, _HID)), _full((1, _HID)),
        ],
        out_specs=pl.BlockSpec((blk, _HID), lambda i: (i, 0)),
        out_shape=jax.ShapeDtypeStruct((_N, _HID), jnp.float32),
    )(h, aggr, eps, W1, b1, g1, be1, W2, b2, ng, nb)


def _pool_body(h, bc, br, gW1, gb1, gg1, gbe1, gW2, gb2,
               hW1, hb1, hg1, hbe1, hW2, hb2, out):
    hh = h[...]
    t = jnp.dot(hh, gW1[...], preferred_element_type=jnp.float32) + gb1[...]
    t = jnp.maximum(t, 0.0)
    t = _ln(t, gg1[...], gbe1[...])
    gate = jnp.dot(t, gW2[...], preferred_element_type=jnp.float32) + gb2[...]

    oh = lax.broadcasted_iota(jnp.int32, (_N, _G), 1) == bc[...]
    ohf = oh.astype(jnp.float32)
    neg = jnp.float32(-jnp.inf)
    gmax = jnp.max(jnp.where(oh, gate, neg), axis=0, keepdims=True)
    gmax = jnp.where(jnp.isfinite(gmax), gmax, 0.0)
    gpn = jnp.sum(ohf * gmax, axis=1, keepdims=True)
    ex = jnp.exp(gate - gpn)
    den = jnp.sum(ohf * ex, axis=0, keepdims=True)
    dpn = jnp.sum(ohf * den, axis=1, keepdims=True)
    attn = ex / (dpn + 1e-16)
    ah = attn * hh

    oht = (lax.broadcasted_iota(jnp.int32, (_G, _N), 0) == br[...]).astype(jnp.float32)
    g = jnp.dot(oht, ah, preferred_element_type=jnp.float32)

    t2 = jnp.dot(g, hW1[...], preferred_element_type=jnp.float32) + hb1[...]
    t2 = jnp.maximum(t2, 0.0)
    t2 = _ln(t2, hg1[...], hbe1[...])
    out[...] = jnp.dot(t2, hW2[...], preferred_element_type=jnp.float32) + hb2[...]


def _pool(h, bc, br, gW1, gb1, gg1, gbe1, gW2, gb2, hW1, hb1, hg1, hbe1, hW2, hb2):
    return pl.pallas_call(
        _pool_body,
        out_shape=jax.ShapeDtypeStruct((_G, 1), jnp.float32),
    )(h, bc, br, gW1, gb1, gg1, gbe1, gW2, gb2, hW1, hb1, hg1, hbe1, hW2, hb2)


def _sc_aggregate(idx4, e, h):
    mesh = plsc.VectorSubcoreMesh(core_axis_name="c", subcore_axis_name="s")

    @functools.partial(
        pl.kernel,
        out_type=jax.ShapeDtypeStruct((2, _R, _HID), jnp.float32),
        mesh=mesh,
        scratch_types=[
            pltpu.VMEM((_BI, 2, _C), jnp.int32),
            pltpu.VMEM((2, _C), jnp.int32),
            pltpu.VMEM((2, _C, _HID // 2), jnp.int32),
            pltpu.VMEM((2, _C, _HID), jnp.float32),
            pltpu.VMEM_SHARED((_R, _HID), jnp.float32),
            pltpu.SemaphoreType.DMA,
            pltpu.SemaphoreType.DMA,
            pltpu.SemaphoreType.DMA,
            pltpu.SemaphoreType.DMA,
            pltpu.SemaphoreType.DMA,
            pltpu.SemaphoreType.DMA,
        ],
    )
    def k(idx_hbm, e_hbm, h_hbm, out_hbm, idxb, dstc, ebuf, hbuf,
          aggr_sh, se0, se1, sh0, sh1, ss0, ss1):
        c = lax.axis_index("c")
        s = lax.axis_index("s")
        wid = s * 2 + c
        sem_e = (se0, se1)
        sem_h = (sh0, sh1)
        sem_s = (ss0, ss1)

        zero = jnp.zeros((16,), jnp.float32)

        def zrow(r, carry):
            for kk in range(8):
                hbuf[0, r, pl.ds(kk * 16, 16)] = zero
            return carry

        lax.fori_loop(0, _C, zrow, 0)
        for b in range(_STRIPE // _C):
            pltpu.sync_copy(hbuf.at[0],
                            aggr_sh.at[pl.ds(s * _STRIPE + b * _C, _C)])
        plsc.subcore_barrier()

        def wait_e(b):
            pltpu.make_async_copy(
                e_hbm.at[pl.ds(0, _C)], ebuf.at[b], sem_e[b]).wait()

        def wait_h(b):
            pltpu.make_async_copy(
                h_hbm.at[idxb.at[0, 0]], hbuf.at[b], sem_h[b]).wait()

        def wait_s(b):
            pltpu.make_async_copy(
                hbuf.at[b], aggr_sh.at[dstc.at[b]], sem_s[b]).wait()

        pltpu.sync_copy(idx_hbm.at[wid, pl.ds(0, _BI)], idxb)
        pltpu.async_copy(e_hbm.at[pl.ds(wid * _EP, _C)], ebuf.at[0], sem_e[0])
        pltpu.async_copy(h_hbm.at[idxb.at[0, 0]], hbuf.at[0], sem_h[0])

        def pair(g, carry):
            for b in range(2):
                nb = 1 - b
                j = g * 2 + b
                jn = j + 1
                jb = 2 * lax.rem(g, _BI // 2) + b

                wait_e(b)
                wait_h(b)

                for kk in range(_C // 16):
                    dstc[b, pl.ds(kk * 16, 16)] = idxb[jb, 1,
                                                       pl.ds(kk * 16, 16)]

                if b == 0:
                    jbn = jb + 1

                    @pl.when(g >= 1)
                    def _():
                        wait_s(nb)

                    pltpu.async_copy(e_hbm.at[pl.ds(wid * _EP + jn * _C, _C)],
                                     ebuf.at[nb], sem_e[nb])
                    pltpu.async_copy(h_hbm.at[idxb.at[jbn, 0]], hbuf.at[nb],
                                     sem_h[nb])
                else:
                    @pl.when(jn < _CH)
                    def _():
                        wait_s(nb)

                        @pl.when(lax.rem(jn, _BI) == 0)
                        def _():
                            pltpu.sync_copy(idx_hbm.at[wid, pl.ds(jn, _BI)],
                                            idxb)

                        jbn = lax.rem(jn, _BI)
                        pltpu.async_copy(
                            e_hbm.at[pl.ds(wid * _EP + jn * _C, _C)],
                            ebuf.at[nb], sem_e[nb])
                        pltpu.async_copy(h_hbm.at[idxb.at[jbn, 0]],
                                         hbuf.at[nb], sem_h[nb])

                mask = jnp.full((16,), -65536, jnp.int32)

                def row(r, rc):
                    for kk in range(4):
                        sl = pl.ds(kk * 16, 16)
                        sh = pl.ds(_HID // 2 + kk * 16, 16)
                        ve = ebuf[b, r, sl]
                        lo = lax.bitcast_convert_type(ve << 16, jnp.float32)
                        hi = lax.bitcast_convert_type(ve & mask, jnp.float32)
                        hbuf[b, r, sl] = jnp.maximum(hbuf[b, r, sl] + lo, 0.0)
                        hbuf[b, r, sh] = jnp.maximum(hbuf[b, r, sh] + hi, 0.0)
                    return rc

                lax.fori_loop(0, _C, row, 0)

                pltpu.async_copy(hbuf.at[b], aggr_sh.at[dstc.at[b]],
                                 sem_s[b], add=True)
            return carry

        lax.fori_loop(0, _CH // 2, pair, 0)
        wait_s(0)
        wait_s(1)
        plsc.subcore_barrier()
        pltpu.sync_copy(aggr_sh.at[pl.ds(s * _STRIPE, _STRIPE)],
                        out_hbm.at[c, pl.ds(s * _STRIPE, _STRIPE)])

    return k(idx4, e, h)


def kernel(x, edge_index, edge_attr, batch, params):
    p = params
    f32 = jnp.float32

    dcol = jnp.ones((_E, 1), f32)
    ea2 = jnp.concatenate([
        jnp.concatenate([edge_attr, dcol], axis=1),
        jnp.concatenate([edge_attr, -dcol], axis=1),
    ], axis=0)
    ea2 = jnp.pad(ea2, ((0, _E2P - 2 * _E), (0, 32 - 17)))
    W1p = jnp.pad(p['edge_W1'], ((0, 32 - 17), (0, 0)))

    pad = _E2P - 2 * _E
    src_pad = (jnp.arange(pad, dtype=jnp.int32) * 37) % _N
    dst_pad = _N + (jnp.arange(pad, dtype=jnp.int32) % (_R - _N))
    src = jnp.concatenate([edge_index[0], edge_index[1], src_pad])
    dst = jnp.concatenate([edge_index[1], edge_index[0], dst_pad])
    idx4 = jnp.stack([src.reshape(_NW, _CH, _C), dst.reshape(_NW, _CH, _C)],
                     axis=2)

    def row(v):
        return v.reshape(1, -1)

    e = _edge_mlp(ea2, W1p, row(p['edge_b1']), row(p['edge_g1']),
                  row(p['edge_be1']), p['edge_W2'], row(p['edge_b2']))

    h = _node_in(x, p['nin_W'], row(p['nin_b']))

    for l in range(3):
        aggr = _sc_aggregate(idx4, e, h)
        h = _node_update(
            h, aggr, p['conv_eps'][l].reshape(1),
            p['conv_W1'][l], row(p['conv_b1'][l]), row(p['conv_g1'][l]),
            row(p['conv_be1'][l]), p['conv_W2'][l], row(p['conv_b2'][l]),
            row(p['norm_g'][l]), row(p['norm_be'][l]))

    logit = _pool(
        h, batch.reshape(_N, 1), batch.reshape(1, _N),
        p['gate_W1'], row(p['gate_b1']), row(p['gate_g1']), row(p['gate_be1']),
        p['gate_W2'], row(p['gate_b2']),
        p['head_W1'], row(p['head_b1']), row(p['head_g1']), row(p['head_be1']),
        p['head_W2'], row(p['head_b2']))
    return logit.reshape(_G)
